# baseline (speedup 1.0000x reference)
import jax
import jax.numpy as jnp
from jax import lax
from jax.experimental import pallas as pl
from jax.experimental.pallas import tpu as pltpu
from jax.experimental.pallas import tpu_sc as plsc

_M, _N = 16384, 100
_NP = 128
_TOTP = _M * _NP
_NC, _NS = 2, 16
_NW = _NC * _NS
_W = _TOTP // _NW
_CHUNK = 8192
_NCHUNK = _W // _CHUNK
_LANES = 16


def _padded_gather_indices():
    key = jax.random.key(42)
    keys = jax.random.split(key, _N)
    perms = jax.vmap(lambda k: jax.random.permutation(k, _M))(keys)
    perms = perms.T.astype(jnp.int32)
    perms_p = jnp.pad(perms, ((0, 0), (0, _NP - _N)))
    col = jnp.arange(_NP, dtype=jnp.int32)[None, :]
    self_p = jnp.arange(_M, dtype=jnp.int32)[:, None] * _NP + col
    return jnp.where(col < _N, perms_p * _NP + col, self_p), self_p


def _sc_body(xf, eff, out_x, out_m, e_v, g_v, x_v, om_v, sem):
    cid = lax.axis_index("c")
    sid = lax.axis_index("s")
    wid = sid * _NC + cid

    def chunk(k, _):
        base = wid * _W + k * _CHUNK
        pltpu.sync_copy(eff.at[pl.ds(base, _CHUNK)], e_v)
        pltpu.sync_copy(xf.at[pl.ds(base, _CHUNK)], x_v)
        pltpu.async_copy(xf.at[e_v], g_v, sem).wait()

        def vec(i, _):
            b = i * _LANES
            xv = x_v[pl.ds(b, _LANES)]
            gv = g_v[pl.ds(b, _LANES)]
            om_v[pl.ds(b, _LANES)] = jnp.where(xv != gv, 1.0, 0.0)
            return 0

        lax.fori_loop(0, _CHUNK // _LANES, vec, 0)
        pltpu.sync_copy(g_v, out_x.at[pl.ds(base, _CHUNK)])
        pltpu.sync_copy(om_v, out_m.at[pl.ds(base, _CHUNK)])
        return 0

    lax.fori_loop(0, _NCHUNK, chunk, 0)


_sc_call = pl.kernel(
    _sc_body,
    out_type=[jax.ShapeDtypeStruct((_TOTP,), jnp.float32),
              jax.ShapeDtypeStruct((_TOTP,), jnp.float32)],
    mesh=plsc.VectorSubcoreMesh(core_axis_name="c", subcore_axis_name="s"),
    scratch_types=[
        pltpu.VMEM((_CHUNK,), jnp.int32),
        pltpu.VMEM((_CHUNK,), jnp.float32),
        pltpu.VMEM((_CHUNK,), jnp.float32),
        pltpu.VMEM((_CHUNK,), jnp.float32),
        pltpu.SemaphoreType.DMA,
    ],
)


def kernel(x, mask):
    gidx_p, self_p = _padded_gather_indices()
    one = mask[0, 0] * 0.0 + 1.0
    izero = (mask[0, 0] * 0.0).astype(jnp.int32)
    x_p = jnp.pad(x, ((0, 0), (0, _NP - _N)))
    m_p = jnp.pad(mask, ((0, 0), (0, _NP - _N)))
    eff = jnp.where(m_p != 0.0, gidx_p, self_p)
    x1 = x_p.reshape(_TOTP) * one
    eff1 = eff.reshape(_TOTP) + izero
    cx1, cm1 = _sc_call(x1, eff1)
    cx = cx1.reshape(_M, _NP)[:, :_N] * one
    cm = cm1.reshape(_M, _NP)[:, :_N] * one
    return cx, cm


# speedup vs baseline: 9.2108x; 9.2108x over previous
import jax
import jax.numpy as jnp
from jax import lax
from jax.experimental import pallas as pl
from jax.experimental.pallas import tpu as pltpu
from jax.experimental.pallas import tpu_sc as plsc

_M, _N = 16384, 100
_NP = 128
_TOTP = _M * _NP
_NC, _NS = 2, 16
_NW = _NC * _NS
_W = _TOTP // _NW
_CHUNK = 8192
_NCHUNK = _W // _CHUNK
_LANES = 16


def _padded_gather_indices():
    key = jax.random.key(42)
    keys = jax.random.split(key, _N)
    perms = jax.vmap(lambda k: jax.random.permutation(k, _M))(keys)
    perms = perms.T.astype(jnp.int32)
    perms_p = jnp.pad(perms, ((0, 0), (0, _NP - _N)))
    col = jnp.arange(_NP, dtype=jnp.int32)[None, :]
    self_p = jnp.arange(_M, dtype=jnp.int32)[:, None] * _NP + col
    return jnp.where(col < _N, perms_p * _NP + col, self_p), self_p


def _sc_body(xf, eff, out_x, out_m, e_v, g_v, x_v, om_v, sem):
    cid = lax.axis_index("c")
    sid = lax.axis_index("s")
    wid = sid * _NC + cid

    def chunk(k, _):
        base = wid * _W + k * _CHUNK
        pltpu.sync_copy(eff.at[pl.ds(base, _CHUNK)], e_v)
        pltpu.sync_copy(xf.at[pl.ds(base, _CHUNK)], x_v)
        pltpu.async_copy(xf.at[e_v], g_v, sem).wait()

        def vec(i, _):
            b = i * _LANES
            xv = x_v[pl.ds(b, _LANES)]
            gv = g_v[pl.ds(b, _LANES)]
            om_v[pl.ds(b, _LANES)] = jnp.where(xv != gv, 1.0, 0.0)
            return 0

        lax.fori_loop(0, _CHUNK // _LANES, vec, 0)
        pltpu.sync_copy(g_v, out_x.at[pl.ds(base, _CHUNK)])
        pltpu.sync_copy(om_v, out_m.at[pl.ds(base, _CHUNK)])
        return 0

    lax.fori_loop(0, _NCHUNK, chunk, 0)


_sc_call = pl.kernel(
    _sc_body,
    out_type=[jax.ShapeDtypeStruct((_TOTP,), jnp.float32),
              jax.ShapeDtypeStruct((_TOTP,), jnp.float32)],
    mesh=plsc.VectorSubcoreMesh(core_axis_name="c", subcore_axis_name="s"),
    scratch_types=[
        pltpu.VMEM((_CHUNK,), jnp.int32),
        pltpu.VMEM((_CHUNK,), jnp.float32),
        pltpu.VMEM((_CHUNK,), jnp.float32),
        pltpu.VMEM((_CHUNK,), jnp.float32),
        pltpu.SemaphoreType.DMA,
    ],
)


_GIDX_P, _SELF_P = _padded_gather_indices()


def kernel(x, mask):
    gidx_p, self_p = _GIDX_P, _SELF_P
    one = mask[0, 0] * 0.0 + 1.0
    izero = (mask[0, 0] * 0.0).astype(jnp.int32)
    x_p = jnp.pad(x, ((0, 0), (0, _NP - _N)))
    m_p = jnp.pad(mask, ((0, 0), (0, _NP - _N)))
    eff = jnp.where(m_p != 0.0, gidx_p, self_p)
    x1 = x_p.reshape(_TOTP) * one
    eff1 = eff.reshape(_TOTP) + izero
    cx1, cm1 = _sc_call(x1, eff1)
    cx = cx1.reshape(_M, _NP)[:, :_N] * one
    cm = cm1.reshape(_M, _NP)[:, :_N] * one
    return cx, cm


# speedup vs baseline: 10.6480x; 1.1560x over previous
import jax
import jax.numpy as jnp
from jax import lax
from jax.experimental import pallas as pl
from jax.experimental.pallas import tpu as pltpu
from jax.experimental.pallas import tpu_sc as plsc

_M, _N = 16384, 100
_NP = 128
_TOTP = _M * _NP
_NC, _NS = 2, 16
_NW = _NC * _NS
_W = _TOTP // _NW
_CHUNK = 8192
_NCHUNK = _W // _CHUNK
_LANES = 16


def _padded_gather_indices():
    key = jax.random.key(42)
    keys = jax.random.split(key, _N)
    perms = jax.vmap(lambda k: jax.random.permutation(k, _M))(keys)
    perms = perms.T.astype(jnp.int32)
    perms_p = jnp.pad(perms, ((0, 0), (0, _NP - _N)))
    col = jnp.arange(_NP, dtype=jnp.int32)[None, :]
    self_p = jnp.arange(_M, dtype=jnp.int32)[:, None] * _NP + col
    return jnp.where(col < _N, perms_p * _NP + col, self_p), self_p


def _sc_body(xf, eff, out_x, out_m, e_v0, e_v1, g_v0, g_v1, x_v, om_v,
             sem0, sem1):
    cid = lax.axis_index("c")
    sid = lax.axis_index("s")
    wid = sid * _NC + cid
    e_bufs, g_bufs, sems = (e_v0, e_v1), (g_v0, g_v1), (sem0, sem1)

    def process(k, gather_copy):
        base = wid * _W + k * _CHUNK
        pltpu.sync_copy(xf.at[pl.ds(base, _CHUNK)], x_v)
        gather_copy.wait()
        g_v = g_bufs[k % 2]

        def vec(i, _):
            b = i * _LANES
            xv = x_v[pl.ds(b, _LANES)]
            gv = g_v[pl.ds(b, _LANES)]
            om_v[pl.ds(b, _LANES)] = jnp.where(xv != gv, 1.0, 0.0)
            return 0

        lax.fori_loop(0, _CHUNK // _LANES, vec, 0)
        pltpu.sync_copy(g_v, out_x.at[pl.ds(base, _CHUNK)])
        pltpu.sync_copy(om_v, out_m.at[pl.ds(base, _CHUNK)])

    prev = None
    for k in range(_NCHUNK):
        base = wid * _W + k * _CHUNK
        pltpu.sync_copy(eff.at[pl.ds(base, _CHUNK)], e_bufs[k % 2])
        g = pltpu.async_copy(xf.at[e_bufs[k % 2]], g_bufs[k % 2], sems[k % 2])
        if prev is not None:
            process(*prev)
        prev = (k, g)
    process(*prev)


_sc_call = pl.kernel(
    _sc_body,
    out_type=[jax.ShapeDtypeStruct((_TOTP,), jnp.float32),
              jax.ShapeDtypeStruct((_TOTP,), jnp.float32)],
    mesh=plsc.VectorSubcoreMesh(core_axis_name="c", subcore_axis_name="s"),
    scratch_types=[
        pltpu.VMEM((_CHUNK,), jnp.int32),
        pltpu.VMEM((_CHUNK,), jnp.int32),
        pltpu.VMEM((_CHUNK,), jnp.float32),
        pltpu.VMEM((_CHUNK,), jnp.float32),
        pltpu.VMEM((_CHUNK,), jnp.float32),
        pltpu.VMEM((_CHUNK,), jnp.float32),
        pltpu.SemaphoreType.DMA,
        pltpu.SemaphoreType.DMA,
    ],
)


_GIDX_P, _SELF_P = _padded_gather_indices()


def kernel(x, mask):
    gidx_p, self_p = _GIDX_P, _SELF_P
    one = mask[0, 0] * 0.0 + 1.0
    izero = (mask[0, 0] * 0.0).astype(jnp.int32)
    x_p = jnp.pad(x, ((0, 0), (0, _NP - _N)))
    m_p = jnp.pad(mask, ((0, 0), (0, _NP - _N)))
    eff = jnp.where(m_p != 0.0, gidx_p, self_p)
    x1 = x_p.reshape(_TOTP) * one
    eff1 = eff.reshape(_TOTP) + izero
    cx1, cm1 = _sc_call(x1, eff1)
    cx = cx1.reshape(_M, _NP)[:, :_N] * one
    cm = cm1.reshape(_M, _NP)[:, :_N] * one
    return cx, cm
